# baseline (speedup 1.0000x reference)
import functools

import jax
import jax.numpy as jnp
from jax import lax
from jax.experimental import pallas as pl
from jax.experimental.pallas import tpu as pltpu
from jax.experimental.pallas import tpu_sc as plsc

_VOCAB = 100000
_HIDDEN = 128
_MASK_ID = 103
_K = 5
_NSLOT = 8

_TB = 256
_VC = 12800
_NEG = -1.0
_BIGI = jnp.iinfo(jnp.int32).max


def _sigmoid(x):
    e = jnp.exp(-jnp.abs(x))
    return jnp.where(x >= 0, 1.0 / (1.0 + e), e / (1.0 + e))


_BIGF = 1.0e9


def _tc_body(xt_ref, p_ref, sa_ref, sb_ref, ss_ref, gi_ref, gw_ref,
             rv, ri, ea, il):
    v = pl.program_id(1)
    nv = pl.num_programs(1)

    @pl.when(v == 0)
    def _init():
        rv[...] = jnp.full((_TB, _NSLOT), _NEG, jnp.float32)
        ri[...] = jnp.full((_TB, _NSLOT), _BIGF, jnp.float32)
        ea[...] = jnp.zeros((_TB, 1), jnp.float32)
        il[...] = lax.broadcasted_iota(jnp.int32, (1, _VC), 1).astype(
            jnp.float32)

    def extract5(cv, ci):
        vs, idxs = [], []
        for _ in range(_K):
            m = jnp.max(cv, axis=1, keepdims=True)
            pos = jnp.min(jnp.where(cv == m, ci, _BIGF), axis=1,
                          keepdims=True)
            vs.append(m)
            idxs.append(pos)
            cv = jnp.where(ci == pos, _NEG, cv)
        return vs, idxs

    def merge(vs, idxs):
        base = jnp.float32(v * _VC)
        pad_v = jnp.full((_TB, 1), _NEG, jnp.float32)
        pad_i = jnp.full((_TB, 1), _BIGF, jnp.float32)
        cat_v = jnp.concatenate([rv[...]] + vs + [pad_v] * (_NSLOT - _K),
                                axis=1)
        cat_i = jnp.concatenate([ri[...]] + [i + base for i in idxs]
                                + [pad_i] * (_NSLOT - _K), axis=1)
        return extract5(cat_v, cat_i)

    def commit(mvs, mis):
        pad_v = jnp.full((_TB, 1), _NEG, jnp.float32)
        pad_i = jnp.full((_TB, 1), _BIGF, jnp.float32)
        rv[...] = jnp.concatenate(mvs + [pad_v] * (_NSLOT - _K), axis=1)
        ri[...] = jnp.concatenate(mis + [pad_i] * (_NSLOT - _K), axis=1)

    def process(w0, lposf, edge):
        pz = jnp.maximum(w0, 0.0) if edge else w0
        lg = jnp.log(jnp.maximum(w0, 1e-30))
        ea[...] -= jnp.sum(pz * lg, axis=1, keepdims=True)

        ng = _VC // 128
        m1 = w0[:, :128]
        p1 = jnp.broadcast_to(lposf[:, :128], (_TB, 128))
        m2 = jnp.full((_TB, 128), _NEG, jnp.float32)
        p2 = jnp.full((_TB, 128), _BIGF, jnp.float32)
        for g in range(1, ng):
            s = w0[:, g * 128:(g + 1) * 128]
            sp = lposf[:, g * 128:(g + 1) * 128]
            b1 = s > m1
            b2 = s > m2
            m2n = jnp.where(b1, m1, jnp.where(b2, s, m2))
            p2n = jnp.where(b1, p1, jnp.where(b2, sp, p2))
            m1 = jnp.where(b1, s, m1)
            p1 = jnp.where(b1, sp, p1)
            m2, p2 = m2n, p2n
        cand_v = jnp.concatenate([m1, m2], axis=1)
        cand_i = jnp.concatenate([p1, p2], axis=1)
        vsf, idf = extract5(cand_v, cand_i)
        mvs, mis = merge(vsf, idf)

        tau = mvs[_K - 1]
        cnt_all = jnp.sum(jnp.where(w0 >= tau, 1.0, 0.0), axis=1,
                          keepdims=True)
        cnt_rep = jnp.sum(jnp.where(cand_v >= tau, 1.0, 0.0), axis=1,
                          keepdims=True)
        bad = jnp.max(cnt_all - cnt_rep)

        @pl.when(bad == 0.0)
        def _fast():
            commit(mvs, mis)

        @pl.when(bad != 0.0)
        def _slow():
            vs2, id2 = extract5(w0, jnp.broadcast_to(lposf, w0.shape))
            mvs2, mis2 = merge(vs2, id2)
            commit(mvs2, mis2)

    @pl.when(v < nv - 1)
    def _full_chunk():
        process(p_ref[...], il[...], False)

    @pl.when(v == nv - 1)
    def _edge_chunk():
        lposf = il[...]
        lim = jnp.float32(_VOCAB - (nv - 1) * _VC)
        process(jnp.where(lposf < lim, p_ref[...], _NEG), lposf, True)

    @pl.when(v == nv - 1)
    def _finalize():
        li = lax.broadcasted_iota(jnp.int32, (_TB, _NSLOT), 1)
        tv = rv[...]
        ti = ri[...].astype(jnp.int32)
        s = jnp.sum(jnp.where(li < _K, tv, 0.0), axis=1, keepdims=True)
        pn = tv / (s + 1e-10)
        a = sa_ref[0, 0]
        b = sb_ref[0, 0]
        sg = ss_ref[0, 0]
        lam = sg * _sigmoid(a * (-ea[...] - b))
        xt = xt_ref[...]
        ism = xt == _MASK_ID
        w_mask = jnp.where(li < _K, lam * pn,
                           jnp.where(li == _K, 1.0 - lam, 0.0))
        w_real = jnp.where(li == 0, 1.0, 0.0)
        gw_ref[...] = jnp.where(ism, w_mask, w_real)
        i_mask = jnp.where(li < _K, ti,
                           jnp.where(li == _K, _MASK_ID, 0))
        i_real = jnp.where(li == 0, xt, 0)
        gi_ref[...] = jnp.where(ism, i_mask, i_real)


def _tc_stage(xt2, p2, sa, sb, sg):
    n = xt2.shape[0]
    nt = n // _TB
    nv = pl.cdiv(_VOCAB, _VC)
    grid = (nt, nv)
    return pl.pallas_call(
        _tc_body,
        grid=grid,
        in_specs=[
            pl.BlockSpec((_TB, 1), lambda t, v: (t, 0)),
            pl.BlockSpec((_TB, _VC), lambda t, v: (t, v)),
            pl.BlockSpec(memory_space=pltpu.SMEM),
            pl.BlockSpec(memory_space=pltpu.SMEM),
            pl.BlockSpec(memory_space=pltpu.SMEM),
        ],
        out_specs=[
            pl.BlockSpec((_TB, _NSLOT), lambda t, v: (t, 0)),
            pl.BlockSpec((_TB, _NSLOT), lambda t, v: (t, 0)),
        ],
        out_shape=[
            jax.ShapeDtypeStruct((n, _NSLOT), jnp.int32),
            jax.ShapeDtypeStruct((n, _NSLOT), jnp.float32),
        ],
        scratch_shapes=[
            pltpu.VMEM((_TB, _NSLOT), jnp.float32),
            pltpu.VMEM((_TB, _NSLOT), jnp.float32),
            pltpu.VMEM((_TB, 1), jnp.float32),
            pltpu.VMEM((1, _VC), jnp.float32),
        ],
        compiler_params=pltpu.CompilerParams(
            dimension_semantics=("parallel", "arbitrary")),
    )(xt2, p2, sa, sb, sg)


def _sc_combine(table, gi_flat, w_r, n):
    info = plsc.get_sparse_core_info()
    nc, ns, nl = info.num_cores, info.num_subcores, info.num_lanes
    nw = nc * ns
    tpw = n // nw
    mesh = plsc.VectorSubcoreMesh(core_axis_name="c", subcore_axis_name="s")

    @functools.partial(
        pl.kernel,
        mesh=mesh,
        out_type=jax.ShapeDtypeStruct((n, _HIDDEN), jnp.float32),
        scratch_types=[
            pltpu.VMEM((_NSLOT * tpw,), jnp.int32),
            pltpu.VMEM((_NSLOT, tpw, _HIDDEN), jnp.float32),
            pltpu.VMEM((_NSLOT, tpw, _HIDDEN), jnp.float32),
            pltpu.VMEM((tpw, _HIDDEN), jnp.float32),
            pltpu.SemaphoreType.DMA,
        ],
    )
    def k(tab_hbm, gi_hbm, w_hbm, out_hbm, idx_v, w_v, rows_v, out_v, sem):
        wid = lax.axis_index("s") * nc + lax.axis_index("c")
        base = wid * tpw
        pltpu.sync_copy(gi_hbm.at[pl.ds(wid * _NSLOT * tpw, _NSLOT * tpw)],
                        idx_v)
        pltpu.sync_copy(w_hbm.at[wid], w_v)
        for s in range(_NSLOT):
            pltpu.async_copy(tab_hbm.at[idx_v.at[pl.ds(s * tpw, tpw)]],
                             rows_v.at[s], sem).wait()

        def body(j, carry):
            for c in range(_HIDDEN // nl):
                sl = pl.ds(c * nl, nl)
                acc = w_v[0, j, sl] * rows_v[0, j, sl]
                for s in range(1, _NSLOT):
                    acc = acc + w_v[s, j, sl] * rows_v[s, j, sl]
                out_v[j, sl] = acc
            return carry

        lax.fori_loop(0, tpw, body, 0)
        pltpu.sync_copy(out_v, out_hbm.at[pl.ds(base, tpw)])

    return k(table, gi_flat, w_r)


def kernel(x_t, probs, embedding_weight, omega_s, omega_a, omega_b):
    bsz, seq = x_t.shape
    n = bsz * seq
    p2 = probs.reshape(n, _VOCAB)
    xt2 = x_t.reshape(n, 1).astype(jnp.int32)
    sg = jax.nn.sigmoid(omega_s).astype(jnp.float32).reshape(1, 1)
    sa = jax.nn.softplus(omega_a).astype(jnp.float32).reshape(1, 1)
    sb = (-jax.nn.softplus(omega_b)).astype(jnp.float32).reshape(1, 1)
    gi, gw = _tc_stage(xt2, p2, sa, sb, sg)
    info = plsc.get_sparse_core_info()
    nw = info.num_cores * info.num_subcores
    tpw = n // nw
    gi_flat = gi.reshape(nw, tpw, _NSLOT).transpose(0, 2, 1).reshape(-1)
    gw_r = gw.reshape(nw, tpw, _NSLOT).transpose(0, 2, 1)
    w_r = jnp.broadcast_to(gw_r[..., None], (nw, _NSLOT, tpw, _HIDDEN))
    out = _sc_combine(embedding_weight, gi_flat, w_r, n)
    return out.reshape(bsz, seq, _HIDDEN)


# speedup vs baseline: 2.1715x; 2.1715x over previous
import functools

import jax
import jax.numpy as jnp
from jax import lax
from jax.experimental import pallas as pl
from jax.experimental.pallas import tpu as pltpu
from jax.experimental.pallas import tpu_sc as plsc

_VOCAB = 100000
_HIDDEN = 128
_MASK_ID = 103
_K = 5
_NSLOT = 8

_TB = 256
_VC = 8192
_NEG = -1.0
_BIGI = jnp.iinfo(jnp.int32).max


def _sigmoid(x):
    e = jnp.exp(-jnp.abs(x))
    return jnp.where(x >= 0, 1.0 / (1.0 + e), e / (1.0 + e))


_BIGF = 1.0e9


def _tc_body(xt_ref, p_ref, sa_ref, sb_ref, ss_ref, gi_ref, gw_ref,
             rv, ri, ea, il):
    v = pl.program_id(1)
    nv = pl.num_programs(1)

    @pl.when(v == 0)
    def _init():
        rv[...] = jnp.full((_TB, _NSLOT), _NEG, jnp.float32)
        ri[...] = jnp.full((_TB, _NSLOT), _BIGF, jnp.float32)
        ea[...] = jnp.zeros((_TB, 1), jnp.float32)
        il[...] = lax.broadcasted_iota(jnp.int32, (1, _VC), 1).astype(
            jnp.float32)

    def extract5(cv, ci):
        vs, idxs = [], []
        for _ in range(_K):
            m = jnp.max(cv, axis=1, keepdims=True)
            pos = jnp.min(jnp.where(cv == m, ci, _BIGF), axis=1,
                          keepdims=True)
            vs.append(m)
            idxs.append(pos)
            cv = jnp.where(ci == pos, _NEG, cv)
        return vs, idxs

    def merge(vs, idxs):
        base = jnp.float32(v * _VC)
        pad_v = jnp.full((_TB, 1), _NEG, jnp.float32)
        pad_i = jnp.full((_TB, 1), _BIGF, jnp.float32)
        cat_v = jnp.concatenate([rv[...]] + vs + [pad_v] * (_NSLOT - _K),
                                axis=1)
        cat_i = jnp.concatenate([ri[...]] + [i + base for i in idxs]
                                + [pad_i] * (_NSLOT - _K), axis=1)
        return extract5(cat_v, cat_i)

    def commit(mvs, mis):
        pad_v = jnp.full((_TB, 1), _NEG, jnp.float32)
        pad_i = jnp.full((_TB, 1), _BIGF, jnp.float32)
        rv[...] = jnp.concatenate(mvs + [pad_v] * (_NSLOT - _K), axis=1)
        ri[...] = jnp.concatenate(mis + [pad_i] * (_NSLOT - _K), axis=1)

    def process(w0, lposf, edge):
        pz = jnp.maximum(w0, 0.0) if edge else w0
        lg = jnp.log(jnp.maximum(w0, 1e-30))
        ea[...] -= jnp.sum(pz * lg, axis=1, keepdims=True)

        ng = _VC // 128
        m1 = w0[:, :128]
        p1 = jnp.broadcast_to(lposf[:, :128], (_TB, 128))
        m2 = jnp.full((_TB, 128), _NEG, jnp.float32)
        p2 = jnp.full((_TB, 128), _BIGF, jnp.float32)
        for g in range(1, ng):
            s = w0[:, g * 128:(g + 1) * 128]
            sp = lposf[:, g * 128:(g + 1) * 128]
            b1 = s > m1
            b2 = s > m2
            m2n = jnp.where(b1, m1, jnp.where(b2, s, m2))
            p2n = jnp.where(b1, p1, jnp.where(b2, sp, p2))
            m1 = jnp.where(b1, s, m1)
            p1 = jnp.where(b1, sp, p1)
            m2, p2 = m2n, p2n
        cand_v = jnp.concatenate([m1, m2], axis=1)
        cand_i = jnp.concatenate([p1, p2], axis=1)
        vsf, idf = extract5(cand_v, cand_i)
        mvs, mis = merge(vsf, idf)

        tau = mvs[_K - 1]
        cnt_all = jnp.sum(jnp.where(w0 >= tau, 1.0, 0.0), axis=1,
                          keepdims=True)
        cnt_rep = jnp.sum(jnp.where(cand_v >= tau, 1.0, 0.0), axis=1,
                          keepdims=True)
        bad = jnp.max(cnt_all - cnt_rep)

        @pl.when(bad == 0.0)
        def _fast():
            commit(mvs, mis)

        @pl.when(bad != 0.0)
        def _slow():
            vs2, id2 = extract5(w0, jnp.broadcast_to(lposf, w0.shape))
            mvs2, mis2 = merge(vs2, id2)
            commit(mvs2, mis2)

    @pl.when(v < nv - 1)
    def _full_chunk():
        process(p_ref[...], il[...], False)

    @pl.when(v == nv - 1)
    def _edge_chunk():
        lposf = il[...]
        lim = jnp.float32(_VOCAB - (nv - 1) * _VC)
        process(jnp.where(lposf < lim, p_ref[...], _NEG), lposf, True)

    @pl.when(v == nv - 1)
    def _finalize():
        li = lax.broadcasted_iota(jnp.int32, (_TB, _NSLOT), 1)
        tv = rv[...]
        ti = ri[...].astype(jnp.int32)
        s = jnp.sum(jnp.where(li < _K, tv, 0.0), axis=1, keepdims=True)
        pn = tv / (s + 1e-10)
        a = sa_ref[0, 0]
        b = sb_ref[0, 0]
        sg = ss_ref[0, 0]
        lam = sg * _sigmoid(a * (-ea[...] - b))
        xt = xt_ref[...]
        ism = xt == _MASK_ID
        w_mask = jnp.where(li < _K, lam * pn,
                           jnp.where(li == _K, 1.0 - lam, 0.0))
        w_real = jnp.where(li == 0, 1.0, 0.0)
        gw_ref[...] = jnp.where(ism, w_mask, w_real)
        i_mask = jnp.where(li < _K, ti,
                           jnp.where(li == _K, _MASK_ID, 0))
        i_real = jnp.where(li == 0, xt, 0)
        gi_ref[...] = jnp.where(ism, i_mask, i_real)


def _tc_stage(xt2, p2, sa, sb, sg):
    n = xt2.shape[0]
    nt = n // _TB
    nv = pl.cdiv(_VOCAB, _VC)
    grid = (nt, nv)
    return pl.pallas_call(
        _tc_body,
        grid=grid,
        in_specs=[
            pl.BlockSpec((_TB, 1), lambda t, v: (t, 0)),
            pl.BlockSpec((_TB, _VC), lambda t, v: (t, v)),
            pl.BlockSpec(memory_space=pltpu.SMEM),
            pl.BlockSpec(memory_space=pltpu.SMEM),
            pl.BlockSpec(memory_space=pltpu.SMEM),
        ],
        out_specs=[
            pl.BlockSpec((_TB, _NSLOT), lambda t, v: (t, 0)),
            pl.BlockSpec((_TB, _NSLOT), lambda t, v: (t, 0)),
        ],
        out_shape=[
            jax.ShapeDtypeStruct((n, _NSLOT), jnp.int32),
            jax.ShapeDtypeStruct((n, _NSLOT), jnp.float32),
        ],
        scratch_shapes=[
            pltpu.VMEM((_TB, _NSLOT), jnp.float32),
            pltpu.VMEM((_TB, _NSLOT), jnp.float32),
            pltpu.VMEM((_TB, 1), jnp.float32),
            pltpu.VMEM((1, _VC), jnp.float32),
        ],
        compiler_params=pltpu.CompilerParams(
            dimension_semantics=("parallel", "arbitrary")),
    )(xt2, p2, sa, sb, sg)


def _sc_combine(table, gi_flat, w_r, n):
    info = plsc.get_sparse_core_info()
    nc, ns, nl = info.num_cores, info.num_subcores, info.num_lanes
    nw = nc * ns
    tpw = n // nw
    mesh = plsc.VectorSubcoreMesh(core_axis_name="c", subcore_axis_name="s")

    @functools.partial(
        pl.kernel,
        mesh=mesh,
        out_type=jax.ShapeDtypeStruct((n, _HIDDEN), jnp.float32),
        scratch_types=[
            pltpu.VMEM((_NSLOT * tpw,), jnp.int32),
            pltpu.VMEM((_NSLOT, tpw, _HIDDEN), jnp.float32),
            pltpu.VMEM((_NSLOT, tpw, _HIDDEN), jnp.float32),
            pltpu.VMEM((tpw, _HIDDEN), jnp.float32),
            pltpu.SemaphoreType.DMA,
        ],
    )
    def k(tab_hbm, gi_hbm, w_hbm, out_hbm, idx_v, w_v, rows_v, out_v, sem):
        wid = lax.axis_index("s") * nc + lax.axis_index("c")
        base = wid * tpw
        pltpu.sync_copy(gi_hbm.at[pl.ds(wid * _NSLOT * tpw, _NSLOT * tpw)],
                        idx_v)
        pltpu.sync_copy(w_hbm.at[wid], w_v)
        for s in range(_NSLOT):
            pltpu.async_copy(tab_hbm.at[idx_v.at[pl.ds(s * tpw, tpw)]],
                             rows_v.at[s], sem).wait()

        def body(j, carry):
            for c in range(_HIDDEN // nl):
                sl = pl.ds(c * nl, nl)
                acc = w_v[0, j, sl] * rows_v[0, j, sl]
                for s in range(1, _NSLOT):
                    acc = acc + w_v[s, j, sl] * rows_v[s, j, sl]
                out_v[j, sl] = acc
            return carry

        lax.fori_loop(0, tpw, body, 0)
        pltpu.sync_copy(out_v, out_hbm.at[pl.ds(base, tpw)])

    return k(table, gi_flat, w_r)


def kernel(x_t, probs, embedding_weight, omega_s, omega_a, omega_b):
    bsz, seq = x_t.shape
    n = bsz * seq
    p2 = probs.reshape(n, _VOCAB)
    xt2 = x_t.reshape(n, 1).astype(jnp.int32)
    sg = jax.nn.sigmoid(omega_s).astype(jnp.float32).reshape(1, 1)
    sa = jax.nn.softplus(omega_a).astype(jnp.float32).reshape(1, 1)
    sb = (-jax.nn.softplus(omega_b)).astype(jnp.float32).reshape(1, 1)
    gi, gw = _tc_stage(xt2, p2, sa, sb, sg)
    info = plsc.get_sparse_core_info()
    nw = info.num_cores * info.num_subcores
    tpw = n // nw
    gi_flat = gi.reshape(nw, tpw, _NSLOT).transpose(0, 2, 1).reshape(-1)
    gw_r = gw.reshape(nw, tpw, _NSLOT).transpose(0, 2, 1)
    w_r = jnp.broadcast_to(gw_r[..., None], (nw, _NSLOT, tpw, _HIDDEN))
    out = _sc_combine(embedding_weight, gi_flat, w_r, n)
    return out.reshape(bsz, seq, _HIDDEN)
